# baseline (speedup 1.0000x reference)
import functools

import jax
import jax.numpy as jnp
from jax import lax
from jax.experimental import pallas as pl
from jax.experimental.pallas import tpu as pltpu
from jax.experimental.pallas import tpu_sc as plsc

_L = 16
_CH = 8192
_NCH = 4
_U = 32
_NACC = 8
_TCB = 32768


def _chunk_sum_reg(buf, accs, nvec):
  accs = list(accs)
  full = nvec // _U

  def body(j, accs):
    accs = list(accs)
    base = j * (_L * _U)
    vs = [buf[pl.ds(base + k * _L, _L)] for k in range(_U)]
    for k in range(_U):
      accs[k % _NACC] = accs[k % _NACC] + vs[k]
    return tuple(accs)

  accs = lax.fori_loop(0, full, body, tuple(accs))
  accs = list(accs)
  for k in range(nvec - full * _U):
    accs[k % _NACC] = accs[k % _NACC] + buf[pl.ds((full * _U + k) * _L, _L)]
  return tuple(accs)


def _tree_sum(accs):
  accs = list(accs)
  while len(accs) > 1:
    accs = [accs[i] + accs[i + 1] for i in range(0, len(accs), 2)]
  return accs[0]


@functools.cache
def _build_sc(off, length):
  info = plsc.get_sparse_core_info()
  nc = info.num_cores
  nw = nc * info.num_subcores
  per_w = _NCH * _CH
  main_elems = nw * per_w
  extra = (length - main_elems) // _CH
  rem = length - main_elems - extra * _CH
  rem_vecs = (rem + _L - 1) // _L
  assert 0 <= extra < nw and off % 8 == 0 and _NCH % 2 == 0
  mesh = plsc.VectorSubcoreMesh(core_axis_name="c", subcore_axis_name="s")

  @functools.partial(
      pl.kernel,
      mesh=mesh,
      out_type=jax.ShapeDtypeStruct((nw, _L), jnp.float32),
      scratch_types=[
          pltpu.VMEM((_CH,), jnp.float32),
          pltpu.VMEM((_CH,), jnp.float32),
          pltpu.VMEM((_CH,), jnp.float32),
          pltpu.VMEM((_L,), jnp.float32),
          pltpu.SemaphoreType.DMA,
          pltpu.SemaphoreType.DMA,
          pltpu.SemaphoreType.DMA,
      ],
  )
  def ksum(vals, out, buf0, buf1, bufx, stage, sem0, sem1, semx):
    wid = lax.axis_index("s") * nc + lax.axis_index("c")
    base = off + wid * per_w
    zero = jnp.zeros((_L,), jnp.float32)
    bufs = (buf0, buf1)
    sems = (sem0, sem1)

    def start(c, b):
      pltpu.async_copy(
          vals.at[pl.ds(base + c * _CH, _CH)], bufs[b], sems[b])

    def wait(b):
      pltpu.make_async_copy(
          vals.at[pl.ds(0, _CH)], bufs[b], sems[b]).wait()

    if extra:
      @pl.when(wid < extra)
      def _():
        pltpu.async_copy(
            vals.at[pl.ds(off + main_elems + wid * _CH, _CH)], bufx, semx)
    if rem:
      @pl.when(wid == nw - 1)
      def _():
        bufx[pl.ds(rem_vecs * _L - _L, _L)] = zero
        pltpu.async_copy(
            vals.at[pl.ds(off + main_elems + extra * _CH, rem)],
            bufx.at[pl.ds(0, rem)], semx)

    start(0, 0)
    start(1, 1)

    def pair_body(p, accs):
      c0 = 2 * p
      wait(0)
      accs = _chunk_sum_reg(buf0, accs, _CH // _L)

      @pl.when(c0 + 2 < _NCH)
      def _():
        start(c0 + 2, 0)

      wait(1)
      accs = _chunk_sum_reg(buf1, accs, _CH // _L)

      @pl.when(c0 + 3 < _NCH)
      def _():
        start(c0 + 3, 1)

      return accs

    accs = lax.fori_loop(0, _NCH // 2, pair_body, (zero,) * _NACC)
    stage[...] = _tree_sum(accs)

    if extra:
      @pl.when(wid < extra)
      def _():
        pltpu.make_async_copy(
            vals.at[pl.ds(0, _CH)], bufx, semx).wait()
        t = _chunk_sum_reg(bufx, (zero,) * _NACC, _CH // _L)
        stage[...] = stage[...] + _tree_sum(t)
    if rem:
      @pl.when(wid == nw - 1)
      def _():
        pltpu.make_async_copy(
            vals.at[pl.ds(0, rem)], bufx.at[pl.ds(0, rem)], semx).wait()
        t = _chunk_sum_reg(bufx, (zero,) * _NACC, rem_vecs)
        stage[...] = stage[...] + _tree_sum(t)

    pltpu.sync_copy(stage, out.at[wid])

  return ksum


def _tc_body(x_ref, out_ref):
  @pl.when(pl.program_id(0) == 0)
  def _():
    out_ref[0, 0] = jnp.float32(0.0)

  out_ref[0, 0] += jnp.sum(x_ref[...])


@functools.cache
def _build_tc(n, length):
  grid = length // _TCB
  return pl.pallas_call(
      _tc_body,
      grid=(grid,),
      in_specs=[pl.BlockSpec((_TCB,), lambda i: (i,))],
      out_specs=pl.BlockSpec(
          (1, 1), lambda i: (0, 0), memory_space=pltpu.SMEM),
      out_shape=jax.ShapeDtypeStruct((1, 1), jnp.float32),
  )


def kernel(values, indices):
  del indices
  n = values.shape[0]
  info = plsc.get_sparse_core_info()
  nw = info.num_cores * info.num_subcores
  sc_min = nw * _NCH * _CH
  tc_len = ((n - sc_min) // _TCB) * _TCB
  sc_partials = _build_sc(tc_len, n - tc_len)(values)
  tc_part = _build_tc(n, tc_len)(values)
  return jnp.sum(sc_partials) + tc_part[0, 0]


# speedup vs baseline: 1.2418x; 1.2418x over previous
import functools

import jax
import jax.numpy as jnp
from jax import lax
from jax.experimental import pallas as pl
from jax.experimental.pallas import tpu as pltpu
from jax.experimental.pallas import tpu_sc as plsc

_L = 16
_CH = 8192
_NCH = 4
_U = 32
_NACC = 8
_TCB = 32768


def _chunk_sum_reg(buf, accs, nvec):
  accs = list(accs)
  full = nvec // _U

  def body(j, accs):
    accs = list(accs)
    base = j * (_L * _U)
    vs = [buf[pl.ds(base + k * _L, _L)] for k in range(_U)]
    for k in range(_U):
      accs[k % _NACC] = accs[k % _NACC] + vs[k]
    return tuple(accs)

  accs = lax.fori_loop(0, full, body, tuple(accs))
  accs = list(accs)
  for k in range(nvec - full * _U):
    accs[k % _NACC] = accs[k % _NACC] + buf[pl.ds((full * _U + k) * _L, _L)]
  return tuple(accs)


def _tree_sum(accs):
  accs = list(accs)
  while len(accs) > 1:
    accs = [accs[i] + accs[i + 1] for i in range(0, len(accs), 2)]
  return accs[0]


@functools.cache
def _build_sc(off, length):
  info = plsc.get_sparse_core_info()
  nc = info.num_cores
  nw = nc * info.num_subcores
  per_w = _NCH * _CH
  main_elems = nw * per_w
  extra = (length - main_elems) // _CH
  rem = length - main_elems - extra * _CH
  rem_vecs = (rem + _L - 1) // _L
  assert 0 <= extra < nw and off % 8 == 0 and _NCH % 2 == 0
  mesh = plsc.VectorSubcoreMesh(core_axis_name="c", subcore_axis_name="s")

  @functools.partial(
      pl.kernel,
      mesh=mesh,
      out_type=jax.ShapeDtypeStruct((nw, _L), jnp.float32),
      scratch_types=[
          pltpu.VMEM((_CH,), jnp.float32),
          pltpu.VMEM((_CH,), jnp.float32),
          pltpu.VMEM((_CH,), jnp.float32),
          pltpu.VMEM((_L,), jnp.float32),
          pltpu.SemaphoreType.DMA,
          pltpu.SemaphoreType.DMA,
          pltpu.SemaphoreType.DMA,
      ],
  )
  def ksum(vals, out, buf0, buf1, bufx, stage, sem0, sem1, semx):
    wid = lax.axis_index("s") * nc + lax.axis_index("c")
    base = off + wid * per_w
    zero = jnp.zeros((_L,), jnp.float32)
    bufs = (buf0, buf1)
    sems = (sem0, sem1)

    def start(c, b):
      pltpu.async_copy(
          vals.at[pl.ds(base + c * _CH, _CH)], bufs[b], sems[b])

    def wait(b):
      pltpu.make_async_copy(
          vals.at[pl.ds(0, _CH)], bufs[b], sems[b]).wait()

    if extra:
      @pl.when(wid < extra)
      def _():
        pltpu.async_copy(
            vals.at[pl.ds(off + main_elems + wid * _CH, _CH)], bufx, semx)
    if rem:
      @pl.when(wid == nw - 1)
      def _():
        bufx[pl.ds(rem_vecs * _L - _L, _L)] = zero
        pltpu.async_copy(
            vals.at[pl.ds(off + main_elems + extra * _CH, rem)],
            bufx.at[pl.ds(0, rem)], semx)

    start(0, 0)
    start(1, 1)

    def pair_body(p, accs):
      c0 = 2 * p
      wait(0)
      accs = _chunk_sum_reg(buf0, accs, _CH // _L)

      @pl.when(c0 + 2 < _NCH)
      def _():
        start(c0 + 2, 0)

      wait(1)
      accs = _chunk_sum_reg(buf1, accs, _CH // _L)

      @pl.when(c0 + 3 < _NCH)
      def _():
        start(c0 + 3, 1)

      return accs

    accs = lax.fori_loop(0, _NCH // 2, pair_body, (zero,) * _NACC)
    stage[...] = _tree_sum(accs)

    if extra:
      @pl.when(wid < extra)
      def _():
        pltpu.make_async_copy(
            vals.at[pl.ds(0, _CH)], bufx, semx).wait()
        t = _chunk_sum_reg(bufx, (zero,) * _NACC, _CH // _L)
        stage[...] = stage[...] + _tree_sum(t)
    if rem:
      @pl.when(wid == nw - 1)
      def _():
        pltpu.make_async_copy(
            vals.at[pl.ds(0, rem)], bufx.at[pl.ds(0, rem)], semx).wait()
        t = _chunk_sum_reg(bufx, (zero,) * _NACC, rem_vecs)
        stage[...] = stage[...] + _tree_sum(t)

    pltpu.sync_copy(stage, out.at[wid])

  return ksum


def _tc_body(x_ref, out_ref):
  @pl.when(pl.program_id(0) == 0)
  def _():
    out_ref[...] = jnp.zeros((8, 128), jnp.float32)

  x = x_ref[...].reshape(_TCB // 128, 128)
  acc = out_ref[...]
  for r in range(_TCB // 1024):
    acc = acc + x[r * 8:(r + 1) * 8, :]
  out_ref[...] = acc


@functools.cache
def _build_tc(n, length):
  grid = length // _TCB
  return pl.pallas_call(
      _tc_body,
      grid=(grid,),
      in_specs=[pl.BlockSpec((_TCB,), lambda i: (i,))],
      out_specs=pl.BlockSpec((8, 128), lambda i: (0, 0)),
      out_shape=jax.ShapeDtypeStruct((8, 128), jnp.float32),
  )


def kernel(values, indices):
  del indices
  n = values.shape[0]
  info = plsc.get_sparse_core_info()
  nw = info.num_cores * info.num_subcores
  sc_min = nw * _NCH * _CH
  tc_len = ((n - sc_min) // _TCB) * _TCB
  sc_partials = _build_sc(tc_len, n - tc_len)(values)
  tc_part = _build_tc(n, tc_len)(values)
  return jnp.sum(sc_partials) + jnp.sum(tc_part)


# speedup vs baseline: 2.9499x; 2.3755x over previous
import functools

import jax
import jax.numpy as jnp
from jax import lax
from jax.experimental import pallas as pl
from jax.experimental.pallas import tpu as pltpu
from jax.experimental.pallas import tpu_sc as plsc

_L = 16
_CH = 8192
_NCH = 4
_U = 32
_NACC = 8
_TCB = 524288


def _chunk_sum_reg(buf, accs, nvec):
  accs = list(accs)
  full = nvec // _U

  def body(j, accs):
    accs = list(accs)
    base = j * (_L * _U)
    vs = [buf[pl.ds(base + k * _L, _L)] for k in range(_U)]
    for k in range(_U):
      accs[k % _NACC] = accs[k % _NACC] + vs[k]
    return tuple(accs)

  accs = lax.fori_loop(0, full, body, tuple(accs))
  accs = list(accs)
  for k in range(nvec - full * _U):
    accs[k % _NACC] = accs[k % _NACC] + buf[pl.ds((full * _U + k) * _L, _L)]
  return tuple(accs)


def _tree_sum(accs):
  accs = list(accs)
  while len(accs) > 1:
    accs = [accs[i] + accs[i + 1] for i in range(0, len(accs), 2)]
  return accs[0]


@functools.cache
def _build_sc(off, length):
  info = plsc.get_sparse_core_info()
  nc = info.num_cores
  nw = nc * info.num_subcores
  per_w = _NCH * _CH
  main_elems = nw * per_w
  extra = (length - main_elems) // _CH
  rem = length - main_elems - extra * _CH
  rem_vecs = (rem + _L - 1) // _L
  assert 0 <= extra < nw and off % 8 == 0 and _NCH % 2 == 0
  mesh = plsc.VectorSubcoreMesh(core_axis_name="c", subcore_axis_name="s")

  @functools.partial(
      pl.kernel,
      mesh=mesh,
      out_type=jax.ShapeDtypeStruct((nw, _L), jnp.float32),
      scratch_types=[
          pltpu.VMEM((_CH,), jnp.float32),
          pltpu.VMEM((_CH,), jnp.float32),
          pltpu.VMEM((_CH,), jnp.float32),
          pltpu.VMEM((_L,), jnp.float32),
          pltpu.SemaphoreType.DMA,
          pltpu.SemaphoreType.DMA,
          pltpu.SemaphoreType.DMA,
      ],
  )
  def ksum(vals, out, buf0, buf1, bufx, stage, sem0, sem1, semx):
    wid = lax.axis_index("s") * nc + lax.axis_index("c")
    base = off + wid * per_w
    zero = jnp.zeros((_L,), jnp.float32)
    bufs = (buf0, buf1)
    sems = (sem0, sem1)

    def start(c, b):
      pltpu.async_copy(
          vals.at[pl.ds(base + c * _CH, _CH)], bufs[b], sems[b])

    def wait(b):
      pltpu.make_async_copy(
          vals.at[pl.ds(0, _CH)], bufs[b], sems[b]).wait()

    if extra:
      @pl.when(wid < extra)
      def _():
        pltpu.async_copy(
            vals.at[pl.ds(off + main_elems + wid * _CH, _CH)], bufx, semx)
    if rem:
      @pl.when(wid == nw - 1)
      def _():
        bufx[pl.ds(rem_vecs * _L - _L, _L)] = zero
        pltpu.async_copy(
            vals.at[pl.ds(off + main_elems + extra * _CH, rem)],
            bufx.at[pl.ds(0, rem)], semx)

    start(0, 0)
    start(1, 1)

    def pair_body(p, accs):
      c0 = 2 * p
      wait(0)
      accs = _chunk_sum_reg(buf0, accs, _CH // _L)

      @pl.when(c0 + 2 < _NCH)
      def _():
        start(c0 + 2, 0)

      wait(1)
      accs = _chunk_sum_reg(buf1, accs, _CH // _L)

      @pl.when(c0 + 3 < _NCH)
      def _():
        start(c0 + 3, 1)

      return accs

    accs = lax.fori_loop(0, _NCH // 2, pair_body, (zero,) * _NACC)
    stage[...] = _tree_sum(accs)

    if extra:
      @pl.when(wid < extra)
      def _():
        pltpu.make_async_copy(
            vals.at[pl.ds(0, _CH)], bufx, semx).wait()
        t = _chunk_sum_reg(bufx, (zero,) * _NACC, _CH // _L)
        stage[...] = stage[...] + _tree_sum(t)
    if rem:
      @pl.when(wid == nw - 1)
      def _():
        pltpu.make_async_copy(
            vals.at[pl.ds(0, rem)], bufx.at[pl.ds(0, rem)], semx).wait()
        t = _chunk_sum_reg(bufx, (zero,) * _NACC, rem_vecs)
        stage[...] = stage[...] + _tree_sum(t)

    pltpu.sync_copy(stage, out.at[wid])

  return ksum


def _tc_body(x_ref, out_ref):
  @pl.when(pl.program_id(0) == 0)
  def _():
    out_ref[...] = jnp.zeros((8, 128), jnp.float32)

  x = x_ref[...].reshape(_TCB // 128, 128)
  acc = out_ref[...]
  for r in range(_TCB // 1024):
    acc = acc + x[r * 8:(r + 1) * 8, :]
  out_ref[...] = acc


@functools.cache
def _build_tc(n, length):
  grid = length // _TCB
  return pl.pallas_call(
      _tc_body,
      grid=(grid,),
      in_specs=[pl.BlockSpec((_TCB,), lambda i: (i,))],
      out_specs=pl.BlockSpec((8, 128), lambda i: (0, 0)),
      out_shape=jax.ShapeDtypeStruct((8, 128), jnp.float32),
  )


def kernel(values, indices):
  del indices
  n = values.shape[0]
  info = plsc.get_sparse_core_info()
  nw = info.num_cores * info.num_subcores
  sc_min = nw * _NCH * _CH
  tc_len = ((n - sc_min) // _TCB) * _TCB
  sc_partials = _build_sc(tc_len, n - tc_len)(values)
  tc_part = _build_tc(n, tc_len)(values)
  return jnp.sum(sc_partials) + jnp.sum(tc_part)


# speedup vs baseline: 3.1143x; 1.0557x over previous
import functools

import jax
import jax.numpy as jnp
from jax import lax
from jax.experimental import pallas as pl
from jax.experimental.pallas import tpu as pltpu
from jax.experimental.pallas import tpu_sc as plsc

_L = 16
_CH = 8192
_NCH = 2
_U = 32
_NACC = 8
_TCB = 524288


def _chunk_sum_reg(buf, accs, nvec):
  accs = list(accs)
  full = nvec // _U

  def body(j, accs):
    accs = list(accs)
    base = j * (_L * _U)
    vs = [buf[pl.ds(base + k * _L, _L)] for k in range(_U)]
    for k in range(_U):
      accs[k % _NACC] = accs[k % _NACC] + vs[k]
    return tuple(accs)

  accs = lax.fori_loop(0, full, body, tuple(accs))
  accs = list(accs)
  for k in range(nvec - full * _U):
    accs[k % _NACC] = accs[k % _NACC] + buf[pl.ds((full * _U + k) * _L, _L)]
  return tuple(accs)


def _tree_sum(accs):
  accs = list(accs)
  while len(accs) > 1:
    accs = [accs[i] + accs[i + 1] for i in range(0, len(accs), 2)]
  return accs[0]


@functools.cache
def _build_sc(off, length):
  info = plsc.get_sparse_core_info()
  nc = info.num_cores
  nw = nc * info.num_subcores
  per_w = _NCH * _CH
  main_elems = nw * per_w
  extra = (length - main_elems) // _CH
  rem = length - main_elems - extra * _CH
  rem_vecs = (rem + _L - 1) // _L
  assert 0 <= extra < nw and off % 8 == 0 and _NCH % 2 == 0
  mesh = plsc.VectorSubcoreMesh(core_axis_name="c", subcore_axis_name="s")

  @functools.partial(
      pl.kernel,
      mesh=mesh,
      out_type=jax.ShapeDtypeStruct((nw, _L), jnp.float32),
      scratch_types=[
          pltpu.VMEM((_CH,), jnp.float32),
          pltpu.VMEM((_CH,), jnp.float32),
          pltpu.VMEM((_CH,), jnp.float32),
          pltpu.VMEM((_L,), jnp.float32),
          pltpu.SemaphoreType.DMA,
          pltpu.SemaphoreType.DMA,
          pltpu.SemaphoreType.DMA,
      ],
  )
  def ksum(vals, out, buf0, buf1, bufx, stage, sem0, sem1, semx):
    wid = lax.axis_index("s") * nc + lax.axis_index("c")
    base = off + wid * per_w
    zero = jnp.zeros((_L,), jnp.float32)
    bufs = (buf0, buf1)
    sems = (sem0, sem1)

    def start(c, b):
      pltpu.async_copy(
          vals.at[pl.ds(base + c * _CH, _CH)], bufs[b], sems[b])

    def wait(b):
      pltpu.make_async_copy(
          vals.at[pl.ds(0, _CH)], bufs[b], sems[b]).wait()

    if extra:
      @pl.when(wid < extra)
      def _():
        pltpu.async_copy(
            vals.at[pl.ds(off + main_elems + wid * _CH, _CH)], bufx, semx)
    if rem:
      @pl.when(wid == nw - 1)
      def _():
        bufx[pl.ds(rem_vecs * _L - _L, _L)] = zero
        pltpu.async_copy(
            vals.at[pl.ds(off + main_elems + extra * _CH, rem)],
            bufx.at[pl.ds(0, rem)], semx)

    start(0, 0)
    start(1, 1)

    def pair_body(p, accs):
      c0 = 2 * p
      wait(0)
      accs = _chunk_sum_reg(buf0, accs, _CH // _L)

      @pl.when(c0 + 2 < _NCH)
      def _():
        start(c0 + 2, 0)

      wait(1)
      accs = _chunk_sum_reg(buf1, accs, _CH // _L)

      @pl.when(c0 + 3 < _NCH)
      def _():
        start(c0 + 3, 1)

      return accs

    accs = lax.fori_loop(0, _NCH // 2, pair_body, (zero,) * _NACC)
    stage[...] = _tree_sum(accs)

    if extra:
      @pl.when(wid < extra)
      def _():
        pltpu.make_async_copy(
            vals.at[pl.ds(0, _CH)], bufx, semx).wait()
        t = _chunk_sum_reg(bufx, (zero,) * _NACC, _CH // _L)
        stage[...] = stage[...] + _tree_sum(t)
    if rem:
      @pl.when(wid == nw - 1)
      def _():
        pltpu.make_async_copy(
            vals.at[pl.ds(0, rem)], bufx.at[pl.ds(0, rem)], semx).wait()
        t = _chunk_sum_reg(bufx, (zero,) * _NACC, rem_vecs)
        stage[...] = stage[...] + _tree_sum(t)

    pltpu.sync_copy(stage, out.at[wid])

  return ksum


_SUB = 32768


@functools.cache
def _build_tc(n, length):
  nch = length // _TCB
  assert nch >= 2

  def body(vals, out, acc, buf0, buf1, sem0, sem1):
    bufs = (buf0, buf1)
    sems = (sem0, sem1)

    def start(c, b):
      pltpu.make_async_copy(
          vals.at[pl.ds(c * _TCB, _TCB)], bufs[b], sems[b]).start()

    def wait(b):
      pltpu.make_async_copy(
          vals.at[pl.ds(0, _TCB)], bufs[b], sems[b]).wait()

    acc[...] = jnp.zeros((8, 128), jnp.float32)
    start(0, 0)
    start(1, 1)
    for c in range(nch):
      b = c % 2

      def sub(j, _, b=b):
        x = bufs[b][pl.ds(j * _SUB, _SUB)].reshape(_SUB // 128, 128)
        parts = []
        for k in range(8):
          t = x[k * 32:k * 32 + 8, :]
          for r in range(1, 4):
            t = t + x[k * 32 + r * 8:k * 32 + (r + 1) * 8, :]
          parts.append(t)
        acc[...] = acc[...] + _tree_sum(parts)
        return 0

      wait(b)
      lax.fori_loop(0, _TCB // _SUB, sub, 0)
      if c + 2 < nch:
        start(c + 2, b)
    out[0, 0] = jnp.sum(acc[...])

  return pl.pallas_call(
      body,
      in_specs=[pl.BlockSpec(memory_space=pl.ANY)],
      out_specs=pl.BlockSpec(memory_space=pltpu.SMEM),
      out_shape=jax.ShapeDtypeStruct((1, 1), jnp.float32),
      scratch_shapes=[
          pltpu.VMEM((8, 128), jnp.float32),
          pltpu.VMEM((_TCB,), jnp.float32),
          pltpu.VMEM((_TCB,), jnp.float32),
          pltpu.SemaphoreType.DMA,
          pltpu.SemaphoreType.DMA,
      ],
  )


def kernel(values, indices):
  del indices
  n = values.shape[0]
  info = plsc.get_sparse_core_info()
  nw = info.num_cores * info.num_subcores
  sc_min = nw * _NCH * _CH
  tc_len = ((n - sc_min) // _TCB) * _TCB
  sc_partials = _build_sc(tc_len, n - tc_len)(values)
  tc_part = _build_tc(n, tc_len)(values)
  return jnp.sum(sc_partials) + tc_part[0, 0]


# speedup vs baseline: 3.1810x; 1.0214x over previous
import functools

import jax
import jax.numpy as jnp
from jax import lax
from jax.experimental import pallas as pl
from jax.experimental.pallas import tpu as pltpu
from jax.experimental.pallas import tpu_sc as plsc

_L = 16
_CH = 8192
_NCH = 2
_U = 32
_NACC = 8
_TCB = 262144
_NBUF = 4


def _chunk_sum_reg(buf, accs, nvec):
  accs = list(accs)
  full = nvec // _U

  def body(j, accs):
    accs = list(accs)
    base = j * (_L * _U)
    vs = [buf[pl.ds(base + k * _L, _L)] for k in range(_U)]
    for k in range(_U):
      accs[k % _NACC] = accs[k % _NACC] + vs[k]
    return tuple(accs)

  accs = lax.fori_loop(0, full, body, tuple(accs))
  accs = list(accs)
  for k in range(nvec - full * _U):
    accs[k % _NACC] = accs[k % _NACC] + buf[pl.ds((full * _U + k) * _L, _L)]
  return tuple(accs)


def _tree_sum(accs):
  accs = list(accs)
  while len(accs) > 1:
    accs = [accs[i] + accs[i + 1] for i in range(0, len(accs), 2)]
  return accs[0]


@functools.cache
def _build_sc(off, length):
  info = plsc.get_sparse_core_info()
  nc = info.num_cores
  nw = nc * info.num_subcores
  per_w = _NCH * _CH
  main_elems = nw * per_w
  extra = (length - main_elems) // _CH
  rem = length - main_elems - extra * _CH
  rem_vecs = (rem + _L - 1) // _L
  assert 0 <= extra < nw and off % 8 == 0 and _NCH % 2 == 0
  mesh = plsc.VectorSubcoreMesh(core_axis_name="c", subcore_axis_name="s")

  @functools.partial(
      pl.kernel,
      mesh=mesh,
      out_type=jax.ShapeDtypeStruct((nw, _L), jnp.float32),
      scratch_types=[
          pltpu.VMEM((_CH,), jnp.float32),
          pltpu.VMEM((_CH,), jnp.float32),
          pltpu.VMEM((_CH,), jnp.float32),
          pltpu.VMEM((_L,), jnp.float32),
          pltpu.SemaphoreType.DMA,
          pltpu.SemaphoreType.DMA,
          pltpu.SemaphoreType.DMA,
      ],
  )
  def ksum(vals, out, buf0, buf1, bufx, stage, sem0, sem1, semx):
    wid = lax.axis_index("s") * nc + lax.axis_index("c")
    base = off + wid * per_w
    zero = jnp.zeros((_L,), jnp.float32)
    bufs = (buf0, buf1)
    sems = (sem0, sem1)

    def start(c, b):
      pltpu.async_copy(
          vals.at[pl.ds(base + c * _CH, _CH)], bufs[b], sems[b])

    def wait(b):
      pltpu.make_async_copy(
          vals.at[pl.ds(0, _CH)], bufs[b], sems[b]).wait()

    if extra:
      @pl.when(wid < extra)
      def _():
        pltpu.async_copy(
            vals.at[pl.ds(off + main_elems + wid * _CH, _CH)], bufx, semx)
    if rem:
      @pl.when(wid == nw - 1)
      def _():
        bufx[pl.ds(rem_vecs * _L - _L, _L)] = zero
        pltpu.async_copy(
            vals.at[pl.ds(off + main_elems + extra * _CH, rem)],
            bufx.at[pl.ds(0, rem)], semx)

    start(0, 0)
    start(1, 1)

    def pair_body(p, accs):
      c0 = 2 * p
      wait(0)
      accs = _chunk_sum_reg(buf0, accs, _CH // _L)

      @pl.when(c0 + 2 < _NCH)
      def _():
        start(c0 + 2, 0)

      wait(1)
      accs = _chunk_sum_reg(buf1, accs, _CH // _L)

      @pl.when(c0 + 3 < _NCH)
      def _():
        start(c0 + 3, 1)

      return accs

    accs = lax.fori_loop(0, _NCH // 2, pair_body, (zero,) * _NACC)
    stage[...] = _tree_sum(accs)

    if extra:
      @pl.when(wid < extra)
      def _():
        pltpu.make_async_copy(
            vals.at[pl.ds(0, _CH)], bufx, semx).wait()
        t = _chunk_sum_reg(bufx, (zero,) * _NACC, _CH // _L)
        stage[...] = stage[...] + _tree_sum(t)
    if rem:
      @pl.when(wid == nw - 1)
      def _():
        pltpu.make_async_copy(
            vals.at[pl.ds(0, rem)], bufx.at[pl.ds(0, rem)], semx).wait()
        t = _chunk_sum_reg(bufx, (zero,) * _NACC, rem_vecs)
        stage[...] = stage[...] + _tree_sum(t)

    pltpu.sync_copy(stage, out.at[wid])

  return ksum


_SUB = 32768


@functools.cache
def _build_tc(n, length):
  nch = length // _TCB
  assert nch >= _NBUF

  def body(vals, out, acc, *rest):
    bufs = rest[:_NBUF]
    sems = rest[_NBUF:]

    def start(c, b):
      pltpu.make_async_copy(
          vals.at[pl.ds(c * _TCB, _TCB)], bufs[b], sems[b]).start()

    def wait(b):
      pltpu.make_async_copy(
          vals.at[pl.ds(0, _TCB)], bufs[b], sems[b]).wait()

    acc[...] = jnp.zeros((8, 128), jnp.float32)
    for b in range(_NBUF):
      start(b, b)
    for c in range(nch):
      b = c % _NBUF

      def sub(j, _, b=b):
        x = bufs[b][pl.ds(j * _SUB, _SUB)].reshape(_SUB // 128, 128)
        parts = []
        for k in range(8):
          t = x[k * 32:k * 32 + 8, :]
          for r in range(1, 4):
            t = t + x[k * 32 + r * 8:k * 32 + (r + 1) * 8, :]
          parts.append(t)
        acc[...] = acc[...] + _tree_sum(parts)
        return 0

      wait(b)
      lax.fori_loop(0, _TCB // _SUB, sub, 0)
      if c + _NBUF < nch:
        start(c + _NBUF, b)
    out[0, 0] = jnp.sum(acc[...])

  return pl.pallas_call(
      body,
      in_specs=[pl.BlockSpec(memory_space=pl.ANY)],
      out_specs=pl.BlockSpec(memory_space=pltpu.SMEM),
      out_shape=jax.ShapeDtypeStruct((1, 1), jnp.float32),
      scratch_shapes=(
          [pltpu.VMEM((8, 128), jnp.float32)]
          + [pltpu.VMEM((_TCB,), jnp.float32) for _ in range(_NBUF)]
          + [pltpu.SemaphoreType.DMA for _ in range(_NBUF)]
      ),
  )


def kernel(values, indices):
  del indices
  n = values.shape[0]
  info = plsc.get_sparse_core_info()
  nw = info.num_cores * info.num_subcores
  sc_min = nw * _NCH * _CH
  tc_len = ((n - sc_min) // _TCB) * _TCB
  sc_partials = _build_sc(tc_len, n - tc_len)(values)
  tc_part = _build_tc(n, tc_len)(values)
  return jnp.sum(sc_partials) + tc_part[0, 0]
